# Initial kernel scaffold; baseline (speedup 1.0000x reference)
import jax
import jax.numpy as jnp
from jax.experimental import pallas as pl


def kernel(x, edge_index, W1, b1, W2, b2):
    raise NotImplementedError("write your pallas kernel here")


# speedup vs baseline: 17.7041x; 17.7041x over previous
import functools

import jax
import jax.numpy as jnp
from jax import lax
from jax.experimental import pallas as pl
from jax.experimental.pallas import tpu as pltpu
from jax.experimental.pallas import tpu_sc as plsc

N = 10000
NP = 10112
E = 320000
EB = 128
NBLK = E // EB
NW = 32
FULL_IT = NBLK // NW
TAIL = NBLK - FULL_IT * NW
RPT = NP // 16
DEG_PAD = 10240
DEG_RPT = DEG_PAD // 16

_MESH = plsc.VectorSubcoreMesh(
    core_axis_name="c", subcore_axis_name="s", num_cores=2, num_subcores=16
)


def _make_prop(feat):
    @functools.partial(
        pl.kernel,
        mesh=_MESH,
        out_type=(jax.ShapeDtypeStruct((NP, feat), jnp.float32),) * 2,
        scratch_types=[
            pltpu.VMEM((EB,), jnp.int32),
            pltpu.VMEM((1, EB), jnp.int32),
            pltpu.VMEM((EB, feat), jnp.float32),
            pltpu.VMEM_SHARED((NP, feat), jnp.float32),
        ],
    )
    def prop(src_hbm, dst_hbm, feat_hbm, zeros_hbm, o0, o1, srcv, dstv, rows, acc):
        c = lax.axis_index("c")
        s = lax.axis_index("s")
        w = c * 16 + s

        pltpu.sync_copy(zeros_hbm, acc.at[pl.ds(s * RPT, RPT)])
        plsc.subcore_barrier()

        def do_block(b):
            pltpu.sync_copy(src_hbm.at[b], srcv)
            pltpu.sync_copy(dst_hbm.at[b], dstv.at[0])
            pltpu.sync_copy(feat_hbm.at[srcv], rows)
            pltpu.sync_copy(rows, acc.at[dstv.at[0]], add=True)

        @pl.loop(0, FULL_IT)
        def _(i):
            do_block(w + i * NW)

        @pl.when(w < TAIL)
        def _():
            do_block(FULL_IT * NW + w)

        plsc.subcore_barrier()

        @pl.when(c == 0)
        def _():
            pltpu.sync_copy(acc.at[pl.ds(s * RPT, RPT)], o0.at[pl.ds(s * RPT, RPT)])

        @pl.when(c == 1)
        def _():
            pltpu.sync_copy(acc.at[pl.ds(s * RPT, RPT)], o1.at[pl.ds(s * RPT, RPT)])

    return prop


_prop128 = _make_prop(128)


@functools.partial(
    pl.kernel,
    mesh=_MESH,
    out_type=(jax.ShapeDtypeStruct((DEG_PAD,), jnp.float32),) * 2,
    scratch_types=[
        pltpu.VMEM((1, EB), jnp.int32),
        pltpu.VMEM((EB,), jnp.float32),
        pltpu.VMEM_SHARED((DEG_PAD,), jnp.float32),
    ],
)
def _deg_kernel(dst_hbm, zeros_hbm, ones_hbm, d0, d1, dstv, onesv, deg):
    c = lax.axis_index("c")
    s = lax.axis_index("s")
    w = c * 16 + s

    pltpu.sync_copy(zeros_hbm, deg.at[pl.ds(s * DEG_RPT, DEG_RPT)])
    pltpu.sync_copy(ones_hbm, onesv)
    plsc.subcore_barrier()

    def do_block(b):
        pltpu.sync_copy(dst_hbm.at[b], dstv.at[0])
        pltpu.sync_copy(onesv, deg.at[dstv.at[0]], add=True)

    @pl.loop(0, FULL_IT)
    def _(i):
        do_block(w + i * NW)

    @pl.when(w < TAIL)
    def _():
        do_block(FULL_IT * NW + w)

    plsc.subcore_barrier()

    @pl.when(c == 0)
    def _():
        pltpu.sync_copy(deg.at[pl.ds(s * DEG_RPT, DEG_RPT)], d0.at[pl.ds(s * DEG_RPT, DEG_RPT)])

    @pl.when(c == 1)
    def _():
        pltpu.sync_copy(deg.at[pl.ds(s * DEG_RPT, DEG_RPT)], d1.at[pl.ds(s * DEG_RPT, DEG_RPT)])


BR = 1264


def _tc1_body(d0_ref, d1_ref, x_ref, dis_ref, dis64_ref, xs_ref):
    deg = 1.0 + d0_ref[...] + d1_ref[...]
    dis = lax.rsqrt(deg)
    dis_b = jnp.broadcast_to(dis, (BR, 128))
    dis_ref[...] = dis_b
    dis64_ref[...] = dis_b[:, :64]
    xs_ref[...] = dis_b * x_ref[...]


def _tc1(d0, d1, x):
    return pl.pallas_call(
        _tc1_body,
        grid=(NP // BR,),
        in_specs=[
            pl.BlockSpec((BR, 1), lambda i: (i, 0)),
            pl.BlockSpec((BR, 1), lambda i: (i, 0)),
            pl.BlockSpec((BR, 128), lambda i: (i, 0)),
        ],
        out_specs=[
            pl.BlockSpec((BR, 128), lambda i: (i, 0)),
            pl.BlockSpec((BR, 64), lambda i: (i, 0)),
            pl.BlockSpec((BR, 128), lambda i: (i, 0)),
        ],
        out_shape=[
            jax.ShapeDtypeStruct((NP, 128), jnp.float32),
            jax.ShapeDtypeStruct((NP, 64), jnp.float32),
            jax.ShapeDtypeStruct((NP, 128), jnp.float32),
        ],
    )(d0, d1, x)


def _tc2_body(dis_ref, p0_ref, p1_ref, xs_ref, w1_ref, b1_ref, w2_ref, out_ref):
    s1 = dis_ref[...] * (p0_ref[...] + p1_ref[...] + xs_ref[...])
    h1 = jnp.dot(s1, w1_ref[...], preferred_element_type=jnp.float32) + b1_ref[...]
    h1 = jnp.maximum(h1, 0.0)
    h2 = jnp.dot(h1, w2_ref[...], preferred_element_type=jnp.float32)
    h2s = dis_ref[:, :64] * h2
    out_ref[...] = jnp.concatenate([h2s, jnp.zeros((BR, 64), jnp.float32)], axis=1)


def _tc2(dis_b, p0, p1, xs, W1, b1, W2):
    return pl.pallas_call(
        _tc2_body,
        grid=(NP // BR,),
        in_specs=[
            pl.BlockSpec((BR, 128), lambda i: (i, 0)),
            pl.BlockSpec((BR, 128), lambda i: (i, 0)),
            pl.BlockSpec((BR, 128), lambda i: (i, 0)),
            pl.BlockSpec((BR, 128), lambda i: (i, 0)),
            pl.BlockSpec((128, 256), lambda i: (0, 0)),
            pl.BlockSpec((1, 256), lambda i: (0, 0)),
            pl.BlockSpec((256, 64), lambda i: (0, 0)),
        ],
        out_specs=pl.BlockSpec((BR, 128), lambda i: (i, 0)),
        out_shape=jax.ShapeDtypeStruct((NP, 128), jnp.float32),
    )(dis_b, p0, p1, xs, W1, b1, W2)


def _tc3_body(dis_ref, q0_ref, q1_ref, h2s_ref, b2_ref, out_ref):
    t = q0_ref[...] + q1_ref[...] + h2s_ref[...]
    o = dis_ref[...] * t[:, :64] + b2_ref[...]
    m = jnp.max(o, axis=1, keepdims=True)
    e = jnp.exp(o - m)
    lse = jnp.log(jnp.sum(e, axis=1, keepdims=True))
    out_ref[...] = o - m - lse


def _tc3(dis_b, q0, q1, h2s, b2):
    return pl.pallas_call(
        _tc3_body,
        grid=(NP // BR,),
        in_specs=[
            pl.BlockSpec((BR, 64), lambda i: (i, 0)),
            pl.BlockSpec((BR, 128), lambda i: (i, 0)),
            pl.BlockSpec((BR, 128), lambda i: (i, 0)),
            pl.BlockSpec((BR, 128), lambda i: (i, 0)),
            pl.BlockSpec((1, 64), lambda i: (0, 0)),
        ],
        out_specs=pl.BlockSpec((BR, 64), lambda i: (i, 0)),
        out_shape=jax.ShapeDtypeStruct((NP, 64), jnp.float32),
    )(dis_b, q0, q1, h2s, b2)


def kernel(x, edge_index, W1, b1, W2, b2):
    ei = edge_index.astype(jnp.int32)
    src2d = ei[0].reshape(NBLK, EB)
    dst2d = ei[1].reshape(NBLK, EB)

    zeros_deg = jnp.zeros((DEG_RPT,), jnp.float32)
    ones_e = jnp.ones((EB,), jnp.float32)
    zeros128 = jnp.zeros((RPT, 128), jnp.float32)

    xp = jnp.pad(x, ((0, NP - N), (0, 0)))
    d0, d1 = _deg_kernel(dst2d, zeros_deg, ones_e)
    dis_b, dis64, xs = _tc1(d0[:NP, None], d1[:NP, None], xp)

    p0, p1 = _prop128(src2d, dst2d, xs, zeros128)
    h2s = _tc2(dis_b, p0, p1, xs, W1, b1[None, :], W2)

    q0, q1 = _prop128(src2d, dst2d, h2s, zeros128)
    return _tc3(dis64, q0, q1, h2s, b2[None, :])[:N]
